# baseline (speedup 1.0000x reference)
import functools

import jax
import jax.numpy as jnp
from jax import lax
from jax.experimental import pallas as pl
from jax.experimental.pallas import tpu as pltpu
from jax.experimental.pallas import tpu_sc as plsc

N_NODES = 50000
N_EDGES = 800000
NPAD = 51200
DUMMY = 50008
BIGV = 1 << 30
NT = 16
EPT = N_EDGES // NT
CH = 2000
NCHUNK = EPT // CH
RL = CH // 16
SLICE = NPAD // NT

_MESH = plsc.VectorSubcoreMesh(core_axis_name="c", subcore_axis_name="s")


def _gelu_exact(x):
    return x * 0.5 * (1.0 + lax.erf(x * 0.7071067811865476))


def _logits_body(ea_ref, w1_ref, b1_ref, w2_ref, b2_ref, w3_ref, b3_ref, o_ref):
    blk, cols = ea_ref.shape[0], ea_ref.shape[1]
    x = ea_ref[...].reshape(blk * cols, 16)
    h = jnp.dot(x, w1_ref[...], preferred_element_type=jnp.float32, precision=lax.Precision.HIGHEST) + b1_ref[...]
    h = _gelu_exact(h)
    h = jnp.dot(h, w2_ref[...], preferred_element_type=jnp.float32, precision=lax.Precision.HIGHEST) + b2_ref[...]
    h = _gelu_exact(h)
    lg = jnp.sum(h * w3_ref[...], axis=1) + b3_ref[0, 0]
    o_ref[...] = lg.reshape(blk, cols)


def _edge_logits(ea_r, W1T, b1r, W2T, b2r, w3r, b3r):
    rows, cols = ea_r.shape[0], ea_r.shape[1]
    blk = 8
    return pl.pallas_call(
        _logits_body,
        grid=(rows // blk,),
        in_specs=[
            pl.BlockSpec((blk, cols, 16), lambda i: (i, 0, 0)),
            pl.BlockSpec((16, 16), lambda i: (0, 0)),
            pl.BlockSpec((1, 16), lambda i: (0, 0)),
            pl.BlockSpec((16, 16), lambda i: (0, 0)),
            pl.BlockSpec((1, 16), lambda i: (0, 0)),
            pl.BlockSpec((1, 16), lambda i: (0, 0)),
            pl.BlockSpec((1, 1), lambda i: (0, 0)),
        ],
        out_specs=pl.BlockSpec((blk, cols), lambda i: (i, 0)),
        out_shape=jax.ShapeDtypeStruct((rows, cols), jnp.float32),
    )(ea_r, W1T, b1r, W2T, b2r, w3r, b3r)


def _matvec_body(c_ref, x_ref, m_ref, o_ref):
    @pl.when(pl.program_id(0) == 0)
    def _():
        o_ref[...] = jnp.zeros_like(o_ref)

    o_ref[...] += jnp.sum(x_ref[...] * m_ref[...] * c_ref[...], axis=0,
                          keepdims=True)


def _weighted_sum(c2d, x32, m32):
    blk = 400
    return pl.pallas_call(
        _matvec_body,
        grid=(N_NODES // blk,),
        in_specs=[
            pl.BlockSpec((blk, 1), lambda i: (i, 0)),
            pl.BlockSpec((blk, 32), lambda i: (i, 0)),
            pl.BlockSpec((blk, 32), lambda i: (i, 0)),
        ],
        out_specs=pl.BlockSpec((1, 32), lambda i: (0, 0)),
        out_shape=jax.ShapeDtypeStruct((1, 32), jnp.float32),
    )(c2d, x32, m32)


CHB = 10000
NCHUNKB = EPT // CHB
RLB = CHB // 16
SB = 4


def _bfs_body(src_hbm, dst_hbm, hop_hbm, t_hbm, hop_out,
              sbuf, dbuf, wbuf, hdbuf, hsbuf, valbuf, tbuf,
              hop_sh):
    tid = lax.axis_index("s")
    nbase = tid * SLICE

    pltpu.sync_copy(hop_hbm.at[pl.ds(nbase, SLICE)], hop_sh.at[pl.ds(nbase, SLICE)])
    pltpu.sync_copy(t_hbm, tbuf)
    plsc.subcore_barrier()
    t = tbuf[...][0]

    for s in range(SB):
        lvl = t + s

        def _chunk(k, _, lvl=lvl):
            eb = tid * EPT + k * CHB
            pltpu.sync_copy(src_hbm.at[pl.ds(eb, CHB)], sbuf)
            pltpu.sync_copy(dst_hbm.at[pl.ds(eb, CHB)], dbuf)
            pltpu.sync_copy(hop_sh.at[dbuf], hdbuf)
            pltpu.sync_copy(hop_sh.at[sbuf], hsbuf)

            def _reg(i, _2):
                vhd = hdbuf[pl.ds(i * 16, 16)]
                vhs = hsbuf[pl.ds(i * 16, 16)]
                vs = sbuf[pl.ds(i * 16, 16)]
                m = (vhd == lvl - 1) & (vhs == BIGV)
                wbuf[pl.ds(i * 16, 16)] = jnp.where(m, vs, DUMMY)
                valbuf[pl.ds(i * 16, 16)] = vhd + 1
                return 0
            lax.fori_loop(0, RLB, _reg, 0)
            pltpu.sync_copy(valbuf, hop_sh.at[wbuf])
            return 0

        lax.fori_loop(0, NCHUNKB, _chunk, 0)
        plsc.subcore_barrier()

    pltpu.sync_copy(hop_sh.at[pl.ds(nbase, SLICE)], hop_out.at[pl.ds(nbase, SLICE)])


_bfs_step = pl.kernel(
    _bfs_body,
    out_type=jax.ShapeDtypeStruct((NPAD,), jnp.int32),
    mesh=_MESH,
    scratch_types=[
        pltpu.VMEM((CHB,), jnp.int32),
        pltpu.VMEM((CHB,), jnp.int32),
        pltpu.VMEM((CHB,), jnp.int32),
        pltpu.VMEM((CHB,), jnp.int32),
        pltpu.VMEM((CHB,), jnp.int32),
        pltpu.VMEM((CHB,), jnp.int32),
        pltpu.VMEM((16,), jnp.int32),
        pltpu.VMEM_SHARED((NPAD,), jnp.int32),
    ],
)


def _alpha_body(src_hbm, dst_hbm, lg_hbm, hop_hbm, vt_hbm, a_hbm,
                sbuf, dbuf, wbuf, hdbuf, hsbuf,
                ewbuf, kvbuf, ebuf, vmbuf, zslice, v8, vmb,
                hop_sh, vmean_sh, k_sh, s_sh):
    tid = lax.axis_index("s")
    cid = lax.axis_index("c")
    nbase = tid * SLICE
    wbase = cid * N_EDGES + tid * EPT

    pltpu.sync_copy(hop_hbm.at[pl.ds(nbase, SLICE)], hop_sh.at[pl.ds(nbase, SLICE)])

    def _fill(i, _):
        zslice[pl.ds(i * 16, 16)] = jnp.zeros((16,), jnp.float32)
        return 0
    lax.fori_loop(0, SLICE // 16, _fill, 0)
    pltpu.sync_copy(zslice, k_sh.at[pl.ds(nbase, SLICE)])
    pltpu.sync_copy(zslice, s_sh.at[pl.ds(nbase, SLICE)])

    pltpu.sync_copy(vt_hbm.at[:, pl.ds(nbase, SLICE)], v8)

    def _vm(i, _):
        acc = jnp.zeros((16,), jnp.float32)
        for l in range(8):
            acc = acc + v8[l, pl.ds(i * 16, 16)]
        vmb[pl.ds(i * 16, 16)] = acc * 0.125
        return 0
    lax.fori_loop(0, SLICE // 16, _vm, 0)
    pltpu.sync_copy(vmb, vmean_sh.at[pl.ds(nbase, SLICE)])
    plsc.subcore_barrier()

    def _c1(k, _):
        eb = tid * EPT + k * CH
        wb = wbase + k * CH
        pltpu.sync_copy(src_hbm.at[pl.ds(eb, CH)], sbuf)
        pltpu.sync_copy(dst_hbm.at[pl.ds(eb, CH)], dbuf)
        pltpu.sync_copy(lg_hbm.at[pl.ds(eb, CH)], ebuf)
        pltpu.sync_copy(hop_sh.at[dbuf], hdbuf)
        pltpu.sync_copy(hop_sh.at[sbuf], hsbuf)
        pltpu.sync_copy(vmean_sh.at[sbuf], vmbuf)

        def _reg(i, _2):
            vhd = hdbuf[pl.ds(i * 16, 16)]
            vhs = hsbuf[pl.ds(i * 16, 16)]
            vd = dbuf[pl.ds(i * 16, 16)]
            ew = ebuf[pl.ds(i * 16, 16)] * vmbuf[pl.ds(i * 16, 16)]
            on = (vhs == vhd + 1) & (vhd != BIGV)
            ewbuf[pl.ds(i * 16, 16)] = jnp.where(on, ew, -1e30)
            wbuf[pl.ds(i * 16, 16)] = jnp.where(on, vd, DUMMY)
            return 0
        lax.fori_loop(0, RL, _reg, 0)
        pltpu.sync_copy(ewbuf, a_hbm.at[pl.ds(wb, CH)])
        pltpu.sync_copy(ewbuf, k_sh.at[wbuf])
        return 0
    lax.fori_loop(0, NCHUNK, _c1, 0)
    plsc.subcore_barrier()

    def _c2(k, _):
        eb = tid * EPT + k * CH
        wb = wbase + k * CH
        pltpu.sync_copy(dst_hbm.at[pl.ds(eb, CH)], dbuf)
        pltpu.sync_copy(a_hbm.at[pl.ds(wb, CH)], ewbuf)
        pltpu.sync_copy(k_sh.at[dbuf], kvbuf)

        def _reg(i, _2):
            ew = ewbuf[pl.ds(i * 16, 16)]
            kv = kvbuf[pl.ds(i * 16, 16)]
            vd = dbuf[pl.ds(i * 16, 16)]
            wbuf[pl.ds(i * 16, 16)] = jnp.where(ew > kv, vd, DUMMY)
            return 0
        lax.fori_loop(0, RL, _reg, 0)
        pltpu.sync_copy(ewbuf, k_sh.at[wbuf])
        return 0
    lax.fori_loop(0, NCHUNK, _c2, 0)
    plsc.subcore_barrier()

    def _c3(k, _):
        eb = tid * EPT + k * CH
        wb = wbase + k * CH
        pltpu.sync_copy(dst_hbm.at[pl.ds(eb, CH)], dbuf)
        pltpu.sync_copy(a_hbm.at[pl.ds(wb, CH)], ewbuf)
        pltpu.sync_copy(k_sh.at[dbuf], kvbuf)

        def _reg(i, _2):
            ew = ewbuf[pl.ds(i * 16, 16)]
            kv = kvbuf[pl.ds(i * 16, 16)]
            ebuf[pl.ds(i * 16, 16)] = jnp.exp(ew - kv)
            return 0
        lax.fori_loop(0, RL, _reg, 0)
        pltpu.sync_copy(ebuf, a_hbm.at[pl.ds(wb, CH)])
        pltpu.sync_copy(ebuf, s_sh.at[dbuf], add=True)
        return 0
    lax.fori_loop(0, NCHUNK, _c3, 0)
    plsc.subcore_barrier()

    def _c4(k, _):
        eb = tid * EPT + k * CH
        wb = wbase + k * CH
        pltpu.sync_copy(dst_hbm.at[pl.ds(eb, CH)], dbuf)
        pltpu.sync_copy(a_hbm.at[pl.ds(wb, CH)], ebuf)
        pltpu.sync_copy(s_sh.at[dbuf], kvbuf)

        def _reg(i, _2):
            e = ebuf[pl.ds(i * 16, 16)]
            sv = kvbuf[pl.ds(i * 16, 16)]
            ewbuf[pl.ds(i * 16, 16)] = e / (sv + 1e-16)
            return 0
        lax.fori_loop(0, RL, _reg, 0)
        pltpu.sync_copy(ewbuf, a_hbm.at[pl.ds(wb, CH)])
        return 0
    lax.fori_loop(0, NCHUNK, _c4, 0)


_alpha_kernel = pl.kernel(
    _alpha_body,
    out_type=jax.ShapeDtypeStruct((2 * N_EDGES,), jnp.float32),
    mesh=_MESH,
    scratch_types=[
        pltpu.VMEM((CH,), jnp.int32),
        pltpu.VMEM((CH,), jnp.int32),
        pltpu.VMEM((CH,), jnp.int32),
        pltpu.VMEM((CH,), jnp.int32),
        pltpu.VMEM((CH,), jnp.int32),
        pltpu.VMEM((CH,), jnp.float32),
        pltpu.VMEM((CH,), jnp.float32),
        pltpu.VMEM((CH,), jnp.float32),
        pltpu.VMEM((CH,), jnp.float32),
        pltpu.VMEM((SLICE,), jnp.float32),
        pltpu.VMEM((8, SLICE), jnp.float32),
        pltpu.VMEM((SLICE,), jnp.float32),
        pltpu.VMEM_SHARED((NPAD,), jnp.int32),
        pltpu.VMEM_SHARED((NPAD,), jnp.float32),
        pltpu.VMEM_SHARED((NPAD,), jnp.float32),
        pltpu.VMEM_SHARED((NPAD,), jnp.float32),
    ],
)


SP = 3


def _prop_body(src_hbm, dst_hbm, a_hbm, c_hbm, c_out,
               sbuf, dbuf, abuf, cvbuf, ubuf, zslice, one16, idx0,
               ca_sh, cb_sh):
    tid = lax.axis_index("s")
    ii = lax.iota(jnp.int32, 16)
    nbase = tid * SLICE

    pltpu.sync_copy(c_hbm.at[pl.ds(nbase, SLICE)], ca_sh.at[pl.ds(nbase, SLICE)])

    def _fill(i, _):
        zslice[pl.ds(i * 16, 16)] = jnp.zeros((16,), jnp.float32)
        return 0
    lax.fori_loop(0, SLICE // 16, _fill, 0)
    one16[...] = jnp.where(ii == 0, 1.0, 0.0).astype(jnp.float32)
    idx0[...] = jnp.where(ii == 0, 0, DUMMY)
    pltpu.sync_copy(zslice, cb_sh.at[pl.ds(nbase, SLICE)])
    plsc.subcore_barrier()

    for p in range(SP):
        src_sh, dst_sh = (ca_sh, cb_sh) if p % 2 == 0 else (cb_sh, ca_sh)

        def _chunk(k, _, src_sh=src_sh, dst_sh=dst_sh):
            eb = tid * EPT + k * CHB
            pltpu.sync_copy(src_hbm.at[pl.ds(eb, CHB)], sbuf)
            pltpu.sync_copy(dst_hbm.at[pl.ds(eb, CHB)], dbuf)
            pltpu.sync_copy(a_hbm.at[pl.ds(eb, CHB)], abuf)
            pltpu.sync_copy(src_sh.at[dbuf], cvbuf)

            def _reg(i, _2):
                a = abuf[pl.ds(i * 16, 16)]
                cv = cvbuf[pl.ds(i * 16, 16)]
                ubuf[pl.ds(i * 16, 16)] = a * cv
                return 0
            lax.fori_loop(0, RLB, _reg, 0)
            pltpu.sync_copy(ubuf, dst_sh.at[sbuf], add=True)
            return 0
        lax.fori_loop(0, NCHUNKB, _chunk, 0)
        plsc.subcore_barrier()
        pltpu.sync_copy(one16, dst_sh.at[idx0])
        if p + 1 < SP:
            pltpu.sync_copy(zslice, src_sh.at[pl.ds(nbase, SLICE)])
        plsc.subcore_barrier()

    out_sh = cb_sh if (SP - 1) % 2 == 0 else ca_sh
    pltpu.sync_copy(out_sh.at[pl.ds(nbase, SLICE)], c_out.at[pl.ds(nbase, SLICE)])


_prop_step = pl.kernel(
    _prop_body,
    out_type=jax.ShapeDtypeStruct((NPAD,), jnp.float32),
    mesh=_MESH,
    scratch_types=[
        pltpu.VMEM((CHB,), jnp.int32),
        pltpu.VMEM((CHB,), jnp.int32),
        pltpu.VMEM((CHB,), jnp.float32),
        pltpu.VMEM((CHB,), jnp.float32),
        pltpu.VMEM((CHB,), jnp.float32),
        pltpu.VMEM((SLICE,), jnp.float32),
        pltpu.VMEM((16,), jnp.float32),
        pltpu.VMEM((16,), jnp.int32),
        pltpu.VMEM_SHARED((NPAD,), jnp.float32),
        pltpu.VMEM_SHARED((NPAD,), jnp.float32),
    ],
)


def kernel(nodes, edge_index, edge_attr, valid, r, fx, W1, b1, W2, b2, W3, b3):
    N = nodes.shape[0]
    src = edge_index[0].astype(jnp.int32)
    dst = edge_index[1].astype(jnp.int32)

    ea_r = edge_attr.reshape(800, 1000, 16)
    logitsE = _edge_logits(
        ea_r, W1.T, b1.reshape(1, 16), W2.T, b2.reshape(1, 16),
        W3.reshape(1, 16), b3.reshape(1, 1)).reshape(N_EDGES)

    vf = valid[0].astype(jnp.float32)
    validT_pad = jnp.zeros((8, NPAD), jnp.float32).at[:, :N].set(vf.T)

    hop0 = jnp.full((NPAD,), BIGV, jnp.int32).at[0].set(0)

    def _bfs_cond(carry):
        _, _, changed = carry
        return changed > 0

    def _bfs_xla(carry):
        hop, t, _ = carry
        t16 = jnp.full((16,), t, jnp.int32)
        hop_new = _bfs_step(src, dst, hop, t16)
        changed = jnp.any(hop_new[:N_NODES] != hop[:N_NODES]).astype(jnp.int32)
        return hop_new, t + SB, changed

    hop, _, _ = lax.while_loop(
        _bfs_cond, _bfs_xla, (hop0, jnp.int32(1), jnp.int32(1)))
    max_hop = jnp.max(jnp.where(hop[:N_NODES] >= BIGV, -1, hop[:N_NODES]))

    alpha = _alpha_kernel(src, dst, logitsE, hop, validT_pad)[:N_EDGES]

    c0 = jnp.zeros((NPAD,), jnp.float32).at[0].set(1.0)

    def _prop_xla(_, c):
        return _prop_step(src, dst, alpha, c)

    n_prop = (max_hop + SP - 1) // SP
    c = lax.fori_loop(0, n_prop, _prop_xla, c0)[:N]

    x32 = nodes.reshape(N, 32)
    m32 = jnp.repeat(vf, 4, axis=1)
    return _weighted_sum(c.reshape(N, 1), x32, m32)


# speedup vs baseline: 3.0142x; 3.0142x over previous
import functools

import jax
import jax.numpy as jnp
from jax import lax
from jax.experimental import pallas as pl
from jax.experimental.pallas import tpu as pltpu
from jax.experimental.pallas import tpu_sc as plsc

N_NODES = 50000
N_EDGES = 800000
NPAD = 51200
DUMMY = 50008
BIGV = 1 << 30
NT = 16
EPT = N_EDGES // NT
CH = 2000
NCHUNK = EPT // CH
RL = CH // 16
SLICE = NPAD // NT

_MESH = plsc.VectorSubcoreMesh(core_axis_name="c", subcore_axis_name="s")


def _gelu_exact(x):
    return x * 0.5 * (1.0 + lax.erf(x * 0.7071067811865476))


def _logits_body(ea_ref, w1_ref, b1_ref, w2_ref, b2_ref, w3_ref, b3_ref, o_ref):
    blk, cols = ea_ref.shape[0], ea_ref.shape[1]
    x = ea_ref[...].reshape(blk * cols, 16)
    h = jnp.dot(x, w1_ref[...], preferred_element_type=jnp.float32, precision=lax.Precision.HIGHEST) + b1_ref[...]
    h = _gelu_exact(h)
    h = jnp.dot(h, w2_ref[...], preferred_element_type=jnp.float32, precision=lax.Precision.HIGHEST) + b2_ref[...]
    h = _gelu_exact(h)
    lg = jnp.sum(h * w3_ref[...], axis=1) + b3_ref[0, 0]
    o_ref[...] = lg.reshape(blk, cols)


def _edge_logits(ea_r, W1T, b1r, W2T, b2r, w3r, b3r):
    rows, cols = ea_r.shape[0], ea_r.shape[1]
    blk = 8
    return pl.pallas_call(
        _logits_body,
        grid=(rows // blk,),
        in_specs=[
            pl.BlockSpec((blk, cols, 16), lambda i: (i, 0, 0)),
            pl.BlockSpec((16, 16), lambda i: (0, 0)),
            pl.BlockSpec((1, 16), lambda i: (0, 0)),
            pl.BlockSpec((16, 16), lambda i: (0, 0)),
            pl.BlockSpec((1, 16), lambda i: (0, 0)),
            pl.BlockSpec((1, 16), lambda i: (0, 0)),
            pl.BlockSpec((1, 1), lambda i: (0, 0)),
        ],
        out_specs=pl.BlockSpec((blk, cols), lambda i: (i, 0)),
        out_shape=jax.ShapeDtypeStruct((rows, cols), jnp.float32),
    )(ea_r, W1T, b1r, W2T, b2r, w3r, b3r)


def _matvec_body(c_ref, x_ref, m_ref, o_ref):
    @pl.when(pl.program_id(0) == 0)
    def _():
        o_ref[...] = jnp.zeros_like(o_ref)

    o_ref[...] += jnp.sum(x_ref[...] * m_ref[...] * c_ref[...], axis=0,
                          keepdims=True)


def _weighted_sum(c2d, x32, m32):
    blk = 400
    return pl.pallas_call(
        _matvec_body,
        grid=(N_NODES // blk,),
        in_specs=[
            pl.BlockSpec((blk, 1), lambda i: (i, 0)),
            pl.BlockSpec((blk, 32), lambda i: (i, 0)),
            pl.BlockSpec((blk, 32), lambda i: (i, 0)),
        ],
        out_specs=pl.BlockSpec((1, 32), lambda i: (0, 0)),
        out_shape=jax.ShapeDtypeStruct((1, 32), jnp.float32),
    )(c2d, x32, m32)


CHB = 10000
NCHUNKB = EPT // CHB
RLB = CHB // 16
SB = 4


def _bfs_body(src_hbm, dst_hbm, hop_hbm, t_hbm, hop_out,
              sbuf, dbuf, wbuf, hdbuf, hsbuf, valbuf, tbuf,
              hop_sh):
    tid = lax.axis_index("s")
    ii = lax.iota(jnp.int32, 16)
    nbase = tid * SLICE

    pltpu.sync_copy(hop_hbm.at[pl.ds(nbase, SLICE)], hop_sh.at[pl.ds(nbase, SLICE)])
    pltpu.sync_copy(t_hbm, tbuf)
    plsc.subcore_barrier()
    t = tbuf[...][0]

    for s in range(SB):
        lvl = t + s

        def _chunk(k, _, lvl=lvl):
            eb = tid * EPT + k * CHB
            pltpu.sync_copy(src_hbm.at[pl.ds(eb, CHB)], sbuf)
            pltpu.sync_copy(dst_hbm.at[pl.ds(eb, CHB)], dbuf)
            pltpu.sync_copy(hop_sh.at[dbuf], hdbuf)
            pltpu.sync_copy(hop_sh.at[sbuf], hsbuf)

            def _reg(i, _2):
                vhd = hdbuf[pl.ds(i * 16, 16)]
                vhs = hsbuf[pl.ds(i * 16, 16)]
                vs = sbuf[pl.ds(i * 16, 16)]
                m = (vhd == lvl - 1) & (vhs == BIGV)
                dummy_v = (50048 + ((i * 16) & 1023)) + ii
                wbuf[pl.ds(i * 16, 16)] = jnp.where(m, vs, dummy_v)
                valbuf[pl.ds(i * 16, 16)] = vhd + 1
                return 0
            lax.fori_loop(0, RLB, _reg, 0)
            pltpu.sync_copy(valbuf, hop_sh.at[wbuf])
            return 0

        lax.fori_loop(0, NCHUNKB, _chunk, 0)
        plsc.subcore_barrier()

    pltpu.sync_copy(hop_sh.at[pl.ds(nbase, SLICE)], hop_out.at[pl.ds(nbase, SLICE)])


_bfs_step = pl.kernel(
    _bfs_body,
    out_type=jax.ShapeDtypeStruct((NPAD,), jnp.int32),
    mesh=_MESH,
    scratch_types=[
        pltpu.VMEM((CHB,), jnp.int32),
        pltpu.VMEM((CHB,), jnp.int32),
        pltpu.VMEM((CHB,), jnp.int32),
        pltpu.VMEM((CHB,), jnp.int32),
        pltpu.VMEM((CHB,), jnp.int32),
        pltpu.VMEM((CHB,), jnp.int32),
        pltpu.VMEM((16,), jnp.int32),
        pltpu.VMEM_SHARED((NPAD,), jnp.int32),
    ],
)


def _alpha_body(src_hbm, dst_hbm, lg_hbm, hop_hbm, vt_hbm, a_hbm,
                sbuf, dbuf, wbuf, hdbuf, hsbuf,
                ewbuf, kvbuf, ebuf, vmbuf, zslice, v8, vmb,
                hop_sh, vmean_sh, k_sh, s_sh):
    tid = lax.axis_index("s")
    cid = lax.axis_index("c")
    ii = lax.iota(jnp.int32, 16)
    nbase = tid * SLICE
    wbase = cid * N_EDGES + tid * EPT

    pltpu.sync_copy(hop_hbm.at[pl.ds(nbase, SLICE)], hop_sh.at[pl.ds(nbase, SLICE)])

    def _fill(i, _):
        zslice[pl.ds(i * 16, 16)] = jnp.zeros((16,), jnp.float32)
        return 0
    lax.fori_loop(0, SLICE // 16, _fill, 0)
    pltpu.sync_copy(zslice, k_sh.at[pl.ds(nbase, SLICE)])
    pltpu.sync_copy(zslice, s_sh.at[pl.ds(nbase, SLICE)])

    pltpu.sync_copy(vt_hbm.at[:, pl.ds(nbase, SLICE)], v8)

    def _vm(i, _):
        acc = jnp.zeros((16,), jnp.float32)
        for l in range(8):
            acc = acc + v8[l, pl.ds(i * 16, 16)]
        vmb[pl.ds(i * 16, 16)] = acc * 0.125
        return 0
    lax.fori_loop(0, SLICE // 16, _vm, 0)
    pltpu.sync_copy(vmb, vmean_sh.at[pl.ds(nbase, SLICE)])
    plsc.subcore_barrier()

    def _c1(k, _):
        eb = tid * EPT + k * CH
        wb = wbase + k * CH
        pltpu.sync_copy(src_hbm.at[pl.ds(eb, CH)], sbuf)
        pltpu.sync_copy(dst_hbm.at[pl.ds(eb, CH)], dbuf)
        pltpu.sync_copy(lg_hbm.at[pl.ds(eb, CH)], ebuf)
        pltpu.sync_copy(hop_sh.at[dbuf], hdbuf)
        pltpu.sync_copy(hop_sh.at[sbuf], hsbuf)
        pltpu.sync_copy(vmean_sh.at[sbuf], vmbuf)

        def _reg(i, _2):
            vhd = hdbuf[pl.ds(i * 16, 16)]
            vhs = hsbuf[pl.ds(i * 16, 16)]
            vd = dbuf[pl.ds(i * 16, 16)]
            ew = ebuf[pl.ds(i * 16, 16)] * vmbuf[pl.ds(i * 16, 16)]
            on = (vhs == vhd + 1) & (vhd != BIGV)
            dummy_v = (50048 + ((i * 16) & 1023)) + ii
            ewbuf[pl.ds(i * 16, 16)] = jnp.where(on, ew, -1e30)
            wbuf[pl.ds(i * 16, 16)] = jnp.where(on, vd, dummy_v)
            return 0
        lax.fori_loop(0, RL, _reg, 0)
        pltpu.sync_copy(ewbuf, a_hbm.at[pl.ds(wb, CH)])
        pltpu.sync_copy(ewbuf, k_sh.at[wbuf])
        return 0
    lax.fori_loop(0, NCHUNK, _c1, 0)
    plsc.subcore_barrier()

    def _c2(k, _):
        eb = tid * EPT + k * CH
        wb = wbase + k * CH
        pltpu.sync_copy(dst_hbm.at[pl.ds(eb, CH)], dbuf)
        pltpu.sync_copy(a_hbm.at[pl.ds(wb, CH)], ewbuf)
        pltpu.sync_copy(k_sh.at[dbuf], kvbuf)

        def _reg(i, _2):
            ew = ewbuf[pl.ds(i * 16, 16)]
            kv = kvbuf[pl.ds(i * 16, 16)]
            vd = dbuf[pl.ds(i * 16, 16)]
            dummy_v = (50048 + ((i * 16) & 1023)) + ii
            wbuf[pl.ds(i * 16, 16)] = jnp.where(ew > kv, vd, dummy_v)
            return 0
        lax.fori_loop(0, RL, _reg, 0)
        pltpu.sync_copy(ewbuf, k_sh.at[wbuf])
        return 0
    lax.fori_loop(0, NCHUNK, _c2, 0)
    plsc.subcore_barrier()

    def _c3(k, _):
        eb = tid * EPT + k * CH
        wb = wbase + k * CH
        pltpu.sync_copy(dst_hbm.at[pl.ds(eb, CH)], dbuf)
        pltpu.sync_copy(a_hbm.at[pl.ds(wb, CH)], ewbuf)
        pltpu.sync_copy(k_sh.at[dbuf], kvbuf)

        def _reg(i, _2):
            ew = ewbuf[pl.ds(i * 16, 16)]
            kv = kvbuf[pl.ds(i * 16, 16)]
            ebuf[pl.ds(i * 16, 16)] = jnp.exp(ew - kv)
            return 0
        lax.fori_loop(0, RL, _reg, 0)
        pltpu.sync_copy(ebuf, a_hbm.at[pl.ds(wb, CH)])
        pltpu.sync_copy(ebuf, s_sh.at[dbuf], add=True)
        return 0
    lax.fori_loop(0, NCHUNK, _c3, 0)
    plsc.subcore_barrier()

    def _c4(k, _):
        eb = tid * EPT + k * CH
        wb = wbase + k * CH
        pltpu.sync_copy(dst_hbm.at[pl.ds(eb, CH)], dbuf)
        pltpu.sync_copy(a_hbm.at[pl.ds(wb, CH)], ebuf)
        pltpu.sync_copy(s_sh.at[dbuf], kvbuf)

        def _reg(i, _2):
            e = ebuf[pl.ds(i * 16, 16)]
            sv = kvbuf[pl.ds(i * 16, 16)]
            ewbuf[pl.ds(i * 16, 16)] = e / (sv + 1e-16)
            return 0
        lax.fori_loop(0, RL, _reg, 0)
        pltpu.sync_copy(ewbuf, a_hbm.at[pl.ds(wb, CH)])
        return 0
    lax.fori_loop(0, NCHUNK, _c4, 0)


_alpha_kernel = pl.kernel(
    _alpha_body,
    out_type=jax.ShapeDtypeStruct((2 * N_EDGES,), jnp.float32),
    mesh=_MESH,
    scratch_types=[
        pltpu.VMEM((CH,), jnp.int32),
        pltpu.VMEM((CH,), jnp.int32),
        pltpu.VMEM((CH,), jnp.int32),
        pltpu.VMEM((CH,), jnp.int32),
        pltpu.VMEM((CH,), jnp.int32),
        pltpu.VMEM((CH,), jnp.float32),
        pltpu.VMEM((CH,), jnp.float32),
        pltpu.VMEM((CH,), jnp.float32),
        pltpu.VMEM((CH,), jnp.float32),
        pltpu.VMEM((SLICE,), jnp.float32),
        pltpu.VMEM((8, SLICE), jnp.float32),
        pltpu.VMEM((SLICE,), jnp.float32),
        pltpu.VMEM_SHARED((NPAD,), jnp.int32),
        pltpu.VMEM_SHARED((NPAD,), jnp.float32),
        pltpu.VMEM_SHARED((NPAD,), jnp.float32),
        pltpu.VMEM_SHARED((NPAD,), jnp.float32),
    ],
)


SP = 3


def _prop_body(src_hbm, dst_hbm, a_hbm, c_hbm, c_out,
               sbuf, dbuf, abuf, cvbuf, ubuf, zslice, one16, idx0,
               ca_sh, cb_sh):
    tid = lax.axis_index("s")
    ii = lax.iota(jnp.int32, 16)
    nbase = tid * SLICE

    pltpu.sync_copy(c_hbm.at[pl.ds(nbase, SLICE)], ca_sh.at[pl.ds(nbase, SLICE)])

    def _fill(i, _):
        zslice[pl.ds(i * 16, 16)] = jnp.zeros((16,), jnp.float32)
        return 0
    lax.fori_loop(0, SLICE // 16, _fill, 0)
    one16[...] = jnp.where(ii == 0, 1.0, 0.0).astype(jnp.float32)
    idx0[...] = jnp.where(ii == 0, 0, DUMMY)
    pltpu.sync_copy(zslice, cb_sh.at[pl.ds(nbase, SLICE)])
    plsc.subcore_barrier()

    for p in range(SP):
        src_sh, dst_sh = (ca_sh, cb_sh) if p % 2 == 0 else (cb_sh, ca_sh)

        def _chunk(k, _, src_sh=src_sh, dst_sh=dst_sh):
            eb = tid * EPT + k * CHB
            pltpu.sync_copy(src_hbm.at[pl.ds(eb, CHB)], sbuf)
            pltpu.sync_copy(dst_hbm.at[pl.ds(eb, CHB)], dbuf)
            pltpu.sync_copy(a_hbm.at[pl.ds(eb, CHB)], abuf)
            pltpu.sync_copy(src_sh.at[dbuf], cvbuf)

            def _reg(i, _2):
                a = abuf[pl.ds(i * 16, 16)]
                cv = cvbuf[pl.ds(i * 16, 16)]
                ubuf[pl.ds(i * 16, 16)] = a * cv
                return 0
            lax.fori_loop(0, RLB, _reg, 0)
            pltpu.sync_copy(ubuf, dst_sh.at[sbuf], add=True)
            return 0
        lax.fori_loop(0, NCHUNKB, _chunk, 0)
        plsc.subcore_barrier()
        pltpu.sync_copy(one16, dst_sh.at[idx0])
        if p + 1 < SP:
            pltpu.sync_copy(zslice, src_sh.at[pl.ds(nbase, SLICE)])
        plsc.subcore_barrier()

    out_sh = cb_sh if (SP - 1) % 2 == 0 else ca_sh
    pltpu.sync_copy(out_sh.at[pl.ds(nbase, SLICE)], c_out.at[pl.ds(nbase, SLICE)])


_prop_step = pl.kernel(
    _prop_body,
    out_type=jax.ShapeDtypeStruct((NPAD,), jnp.float32),
    mesh=_MESH,
    scratch_types=[
        pltpu.VMEM((CHB,), jnp.int32),
        pltpu.VMEM((CHB,), jnp.int32),
        pltpu.VMEM((CHB,), jnp.float32),
        pltpu.VMEM((CHB,), jnp.float32),
        pltpu.VMEM((CHB,), jnp.float32),
        pltpu.VMEM((SLICE,), jnp.float32),
        pltpu.VMEM((16,), jnp.float32),
        pltpu.VMEM((16,), jnp.int32),
        pltpu.VMEM_SHARED((NPAD,), jnp.float32),
        pltpu.VMEM_SHARED((NPAD,), jnp.float32),
    ],
)


def kernel(nodes, edge_index, edge_attr, valid, r, fx, W1, b1, W2, b2, W3, b3):
    N = nodes.shape[0]
    src = edge_index[0].astype(jnp.int32)
    dst = edge_index[1].astype(jnp.int32)

    ea_r = edge_attr.reshape(800, 1000, 16)
    logitsE = _edge_logits(
        ea_r, W1.T, b1.reshape(1, 16), W2.T, b2.reshape(1, 16),
        W3.reshape(1, 16), b3.reshape(1, 1)).reshape(N_EDGES)

    vf = valid[0].astype(jnp.float32)
    validT_pad = jnp.zeros((8, NPAD), jnp.float32).at[:, :N].set(vf.T)

    hop0 = jnp.full((NPAD,), BIGV, jnp.int32).at[0].set(0)

    def _bfs_cond(carry):
        _, _, cont = carry
        return cont > 0

    def _bfs_xla(carry):
        hop, t, _ = carry
        t16 = jnp.full((16,), t, jnp.int32)
        hop_new = _bfs_step(src, dst, hop, t16)
        cur = jnp.max(jnp.where(hop_new[:N_NODES] >= BIGV, -1, hop_new[:N_NODES]))
        cont = (cur == t + SB - 1).astype(jnp.int32)
        return hop_new, t + SB, cont

    hop, _, _ = lax.while_loop(
        _bfs_cond, _bfs_xla, (hop0, jnp.int32(1), jnp.int32(1)))
    max_hop = jnp.max(jnp.where(hop[:N_NODES] >= BIGV, -1, hop[:N_NODES]))

    alpha = _alpha_kernel(src, dst, logitsE, hop, validT_pad)[:N_EDGES]

    c0 = jnp.zeros((NPAD,), jnp.float32).at[0].set(1.0)

    def _prop_xla(_, c):
        return _prop_step(src, dst, alpha, c)

    n_prop = (max_hop + SP - 1) // SP
    c = lax.fori_loop(0, n_prop, _prop_xla, c0)[:N]

    x32 = nodes.reshape(N, 32)
    m32 = jnp.repeat(vf, 4, axis=1)
    return _weighted_sum(c.reshape(N, 1), x32, m32)


# speedup vs baseline: 3.1579x; 1.0477x over previous
import functools

import jax
import jax.numpy as jnp
from jax import lax
from jax.experimental import pallas as pl
from jax.experimental.pallas import tpu as pltpu
from jax.experimental.pallas import tpu_sc as plsc

N_NODES = 50000
N_EDGES = 800000
NPAD = 51200
DUMMY = 50008
BIGV = 1 << 30
NT = 16
EPT = N_EDGES // NT
CH = 2000
NCHUNK = EPT // CH
RL = CH // 16
SLICE = NPAD // NT

_MESH = plsc.VectorSubcoreMesh(core_axis_name="c", subcore_axis_name="s")


def _gelu_exact(x):
    return x * 0.5 * (1.0 + lax.erf(x * 0.7071067811865476))


def _logits_body(ea_ref, w1_ref, b1_ref, w2_ref, b2_ref, w3_ref, b3_ref, o_ref):
    blk, cols = ea_ref.shape[0], ea_ref.shape[1]
    x = ea_ref[...].reshape(blk * cols, 16)
    h = jnp.dot(x, w1_ref[...], preferred_element_type=jnp.float32, precision=lax.Precision.HIGHEST) + b1_ref[...]
    h = _gelu_exact(h)
    h = jnp.dot(h, w2_ref[...], preferred_element_type=jnp.float32, precision=lax.Precision.HIGHEST) + b2_ref[...]
    h = _gelu_exact(h)
    lg = jnp.sum(h * w3_ref[...], axis=1) + b3_ref[0, 0]
    o_ref[...] = lg.reshape(blk, cols)


def _edge_logits(ea_r, W1T, b1r, W2T, b2r, w3r, b3r):
    rows, cols = ea_r.shape[0], ea_r.shape[1]
    blk = 8
    return pl.pallas_call(
        _logits_body,
        grid=(rows // blk,),
        in_specs=[
            pl.BlockSpec((blk, cols, 16), lambda i: (i, 0, 0)),
            pl.BlockSpec((16, 16), lambda i: (0, 0)),
            pl.BlockSpec((1, 16), lambda i: (0, 0)),
            pl.BlockSpec((16, 16), lambda i: (0, 0)),
            pl.BlockSpec((1, 16), lambda i: (0, 0)),
            pl.BlockSpec((1, 16), lambda i: (0, 0)),
            pl.BlockSpec((1, 1), lambda i: (0, 0)),
        ],
        out_specs=pl.BlockSpec((blk, cols), lambda i: (i, 0)),
        out_shape=jax.ShapeDtypeStruct((rows, cols), jnp.float32),
    )(ea_r, W1T, b1r, W2T, b2r, w3r, b3r)


def _matvec_body(c_ref, x_ref, m_ref, o_ref):
    @pl.when(pl.program_id(0) == 0)
    def _():
        o_ref[...] = jnp.zeros_like(o_ref)

    o_ref[...] += jnp.sum(x_ref[...] * m_ref[...] * c_ref[...], axis=0,
                          keepdims=True)


def _weighted_sum(c2d, x32, m32):
    blk = 400
    return pl.pallas_call(
        _matvec_body,
        grid=(N_NODES // blk,),
        in_specs=[
            pl.BlockSpec((blk, 1), lambda i: (i, 0)),
            pl.BlockSpec((blk, 32), lambda i: (i, 0)),
            pl.BlockSpec((blk, 32), lambda i: (i, 0)),
        ],
        out_specs=pl.BlockSpec((1, 32), lambda i: (0, 0)),
        out_shape=jax.ShapeDtypeStruct((1, 32), jnp.float32),
    )(c2d, x32, m32)


CHB = 10000
NCHUNKB = EPT // CHB
RLB = CHB // 16
SB = 4


def _bfs_body(src_hbm, dst_hbm, hop_hbm, t_hbm, hop_out,
              sbuf, dbuf, wbuf, hdbuf, hsbuf, valbuf, tbuf,
              hop_sh):
    tid = lax.axis_index("s")
    ii = lax.iota(jnp.int32, 16)
    nbase = tid * SLICE

    pltpu.sync_copy(hop_hbm.at[pl.ds(nbase, SLICE)], hop_sh.at[pl.ds(nbase, SLICE)])
    pltpu.sync_copy(t_hbm, tbuf)
    plsc.subcore_barrier()
    t = tbuf[...][0]

    for s in range(SB):
        lvl = t + s

        def _chunk(k, _, lvl=lvl):
            eb = tid * EPT + k * CHB
            pltpu.sync_copy(src_hbm.at[pl.ds(eb, CHB)], sbuf)
            pltpu.sync_copy(dst_hbm.at[pl.ds(eb, CHB)], dbuf)
            pltpu.sync_copy(hop_sh.at[dbuf], hdbuf)
            pltpu.sync_copy(hop_sh.at[sbuf], hsbuf)

            def _reg(i, _2):
                vhd = hdbuf[pl.ds(i * 16, 16)]
                vhs = hsbuf[pl.ds(i * 16, 16)]
                vs = sbuf[pl.ds(i * 16, 16)]
                m = (vhd == lvl - 1) & (vhs == BIGV)
                dummy_v = (50048 + ((i * 16) & 1023)) + ii
                wbuf[pl.ds(i * 16, 16)] = jnp.where(m, vs, dummy_v)
                valbuf[pl.ds(i * 16, 16)] = vhd + 1
                return 0
            lax.fori_loop(0, RLB, _reg, 0)
            pltpu.sync_copy(valbuf, hop_sh.at[wbuf])
            return 0

        lax.fori_loop(0, NCHUNKB, _chunk, 0)
        plsc.subcore_barrier()

    pltpu.sync_copy(hop_sh.at[pl.ds(nbase, SLICE)], hop_out.at[pl.ds(nbase, SLICE)])


_bfs_step = pl.kernel(
    _bfs_body,
    out_type=jax.ShapeDtypeStruct((NPAD,), jnp.int32),
    mesh=_MESH,
    scratch_types=[
        pltpu.VMEM((CHB,), jnp.int32),
        pltpu.VMEM((CHB,), jnp.int32),
        pltpu.VMEM((CHB,), jnp.int32),
        pltpu.VMEM((CHB,), jnp.int32),
        pltpu.VMEM((CHB,), jnp.int32),
        pltpu.VMEM((CHB,), jnp.int32),
        pltpu.VMEM((16,), jnp.int32),
        pltpu.VMEM_SHARED((NPAD,), jnp.int32),
    ],
)


def _alpha_body(src_hbm, dst_hbm, lg_hbm, hop_hbm, vt_hbm, a_hbm,
                sbuf, dbuf, wbuf, hdbuf, hsbuf,
                ewbuf, kvbuf, ebuf, vmbuf, zslice, v8, vmb,
                hop_sh, vmean_sh, k_sh, s_sh):
    tid = lax.axis_index("s")
    cid = lax.axis_index("c")
    ii = lax.iota(jnp.int32, 16)
    nbase = tid * SLICE
    wbase = cid * N_EDGES + tid * EPT

    pltpu.sync_copy(hop_hbm.at[pl.ds(nbase, SLICE)], hop_sh.at[pl.ds(nbase, SLICE)])

    def _fill(i, _):
        zslice[pl.ds(i * 16, 16)] = jnp.zeros((16,), jnp.float32)
        return 0
    lax.fori_loop(0, SLICE // 16, _fill, 0)
    pltpu.sync_copy(zslice, k_sh.at[pl.ds(nbase, SLICE)])
    pltpu.sync_copy(zslice, s_sh.at[pl.ds(nbase, SLICE)])

    def _vmz(i, _):
        vmb[pl.ds(i * 16, 16)] = jnp.zeros((16,), jnp.float32)
        return 0
    lax.fori_loop(0, SLICE // 16, _vmz, 0)
    for l in range(8):
        pltpu.sync_copy(vt_hbm.at[l, pl.ds(nbase, SLICE)], v8)

        def _vm(i, _):
            vmb[pl.ds(i * 16, 16)] = vmb[pl.ds(i * 16, 16)] + v8[pl.ds(i * 16, 16)]
            return 0
        lax.fori_loop(0, SLICE // 16, _vm, 0)

    def _vms(i, _):
        vmb[pl.ds(i * 16, 16)] = vmb[pl.ds(i * 16, 16)] * 0.125
        return 0
    lax.fori_loop(0, SLICE // 16, _vms, 0)
    pltpu.sync_copy(vmb, vmean_sh.at[pl.ds(nbase, SLICE)])
    plsc.subcore_barrier()

    def _c1(k, _):
        eb = tid * EPT + k * CHB
        wb = wbase + k * CHB
        pltpu.sync_copy(src_hbm.at[pl.ds(eb, CHB)], sbuf)
        pltpu.sync_copy(dst_hbm.at[pl.ds(eb, CHB)], dbuf)
        pltpu.sync_copy(lg_hbm.at[pl.ds(eb, CHB)], ebuf)
        pltpu.sync_copy(hop_sh.at[dbuf], hdbuf)
        pltpu.sync_copy(hop_sh.at[sbuf], hsbuf)
        pltpu.sync_copy(vmean_sh.at[sbuf], vmbuf)

        def _reg(i, _2):
            vhd = hdbuf[pl.ds(i * 16, 16)]
            vhs = hsbuf[pl.ds(i * 16, 16)]
            vd = dbuf[pl.ds(i * 16, 16)]
            ew = ebuf[pl.ds(i * 16, 16)] * vmbuf[pl.ds(i * 16, 16)]
            on = (vhs == vhd + 1) & (vhd != BIGV)
            dummy_v = (50048 + ((i * 16) & 1023)) + ii
            ewbuf[pl.ds(i * 16, 16)] = jnp.where(on, ew, -1e30)
            wbuf[pl.ds(i * 16, 16)] = jnp.where(on, vd, dummy_v)
            return 0
        lax.fori_loop(0, RLB, _reg, 0)
        pltpu.sync_copy(ewbuf, a_hbm.at[pl.ds(wb, CHB)])
        pltpu.sync_copy(ewbuf, k_sh.at[wbuf])
        return 0
    lax.fori_loop(0, NCHUNKB, _c1, 0)
    plsc.subcore_barrier()

    def _c2(k, _):
        eb = tid * EPT + k * CHB
        wb = wbase + k * CHB
        pltpu.sync_copy(dst_hbm.at[pl.ds(eb, CHB)], dbuf)
        pltpu.sync_copy(a_hbm.at[pl.ds(wb, CHB)], ewbuf)
        pltpu.sync_copy(k_sh.at[dbuf], kvbuf)

        def _reg(i, _2):
            ew = ewbuf[pl.ds(i * 16, 16)]
            kv = kvbuf[pl.ds(i * 16, 16)]
            vd = dbuf[pl.ds(i * 16, 16)]
            dummy_v = (50048 + ((i * 16) & 1023)) + ii
            wbuf[pl.ds(i * 16, 16)] = jnp.where(ew > kv, vd, dummy_v)
            return 0
        lax.fori_loop(0, RLB, _reg, 0)
        pltpu.sync_copy(ewbuf, k_sh.at[wbuf])
        return 0
    lax.fori_loop(0, NCHUNKB, _c2, 0)
    plsc.subcore_barrier()

    def _c3(k, _):
        eb = tid * EPT + k * CHB
        wb = wbase + k * CHB
        pltpu.sync_copy(dst_hbm.at[pl.ds(eb, CHB)], dbuf)
        pltpu.sync_copy(a_hbm.at[pl.ds(wb, CHB)], ewbuf)
        pltpu.sync_copy(k_sh.at[dbuf], kvbuf)

        def _reg(i, _2):
            ew = ewbuf[pl.ds(i * 16, 16)]
            kv = kvbuf[pl.ds(i * 16, 16)]
            ebuf[pl.ds(i * 16, 16)] = jnp.exp(ew - kv)
            return 0
        lax.fori_loop(0, RLB, _reg, 0)
        pltpu.sync_copy(ebuf, a_hbm.at[pl.ds(wb, CHB)])
        pltpu.sync_copy(ebuf, s_sh.at[dbuf], add=True)
        return 0
    lax.fori_loop(0, NCHUNKB, _c3, 0)
    plsc.subcore_barrier()

    def _c4(k, _):
        eb = tid * EPT + k * CHB
        wb = wbase + k * CHB
        pltpu.sync_copy(dst_hbm.at[pl.ds(eb, CHB)], dbuf)
        pltpu.sync_copy(a_hbm.at[pl.ds(wb, CHB)], ebuf)
        pltpu.sync_copy(s_sh.at[dbuf], kvbuf)

        def _reg(i, _2):
            e = ebuf[pl.ds(i * 16, 16)]
            sv = kvbuf[pl.ds(i * 16, 16)]
            ewbuf[pl.ds(i * 16, 16)] = e / (sv + 1e-16)
            return 0
        lax.fori_loop(0, RLB, _reg, 0)
        pltpu.sync_copy(ewbuf, a_hbm.at[pl.ds(wb, CHB)])
        return 0
    lax.fori_loop(0, NCHUNKB, _c4, 0)


_alpha_kernel = pl.kernel(
    _alpha_body,
    out_type=jax.ShapeDtypeStruct((2 * N_EDGES,), jnp.float32),
    mesh=_MESH,
    scratch_types=[
        pltpu.VMEM((CHB,), jnp.int32),
        pltpu.VMEM((CHB,), jnp.int32),
        pltpu.VMEM((CHB,), jnp.int32),
        pltpu.VMEM((CHB,), jnp.int32),
        pltpu.VMEM((CHB,), jnp.int32),
        pltpu.VMEM((CHB,), jnp.float32),
        pltpu.VMEM((CHB,), jnp.float32),
        pltpu.VMEM((CHB,), jnp.float32),
        pltpu.VMEM((CHB,), jnp.float32),
        pltpu.VMEM((SLICE,), jnp.float32),
        pltpu.VMEM((SLICE,), jnp.float32),
        pltpu.VMEM((SLICE,), jnp.float32),
        pltpu.VMEM_SHARED((NPAD,), jnp.int32),
        pltpu.VMEM_SHARED((NPAD,), jnp.float32),
        pltpu.VMEM_SHARED((NPAD,), jnp.float32),
        pltpu.VMEM_SHARED((NPAD,), jnp.float32),
    ],
)


SP = 3


def _prop_body(src_hbm, dst_hbm, a_hbm, c_hbm, c_out,
               sbuf, dbuf, abuf, cvbuf, ubuf, zslice, one16, idx0,
               ca_sh, cb_sh):
    tid = lax.axis_index("s")
    ii = lax.iota(jnp.int32, 16)
    nbase = tid * SLICE

    pltpu.sync_copy(c_hbm.at[pl.ds(nbase, SLICE)], ca_sh.at[pl.ds(nbase, SLICE)])

    def _fill(i, _):
        zslice[pl.ds(i * 16, 16)] = jnp.zeros((16,), jnp.float32)
        return 0
    lax.fori_loop(0, SLICE // 16, _fill, 0)
    one16[...] = jnp.where(ii == 0, 1.0, 0.0).astype(jnp.float32)
    idx0[...] = jnp.where(ii == 0, 0, DUMMY)
    pltpu.sync_copy(zslice, cb_sh.at[pl.ds(nbase, SLICE)])
    plsc.subcore_barrier()

    for p in range(SP):
        src_sh, dst_sh = (ca_sh, cb_sh) if p % 2 == 0 else (cb_sh, ca_sh)

        def _chunk(k, _, src_sh=src_sh, dst_sh=dst_sh):
            eb = tid * EPT + k * CHB
            pltpu.sync_copy(src_hbm.at[pl.ds(eb, CHB)], sbuf)
            pltpu.sync_copy(dst_hbm.at[pl.ds(eb, CHB)], dbuf)
            pltpu.sync_copy(a_hbm.at[pl.ds(eb, CHB)], abuf)
            pltpu.sync_copy(src_sh.at[dbuf], cvbuf)

            def _reg(i, _2):
                a = abuf[pl.ds(i * 16, 16)]
                cv = cvbuf[pl.ds(i * 16, 16)]
                ubuf[pl.ds(i * 16, 16)] = a * cv
                return 0
            lax.fori_loop(0, RLB, _reg, 0)
            pltpu.sync_copy(ubuf, dst_sh.at[sbuf], add=True)
            return 0
        lax.fori_loop(0, NCHUNKB, _chunk, 0)
        plsc.subcore_barrier()
        pltpu.sync_copy(one16, dst_sh.at[idx0])
        if p + 1 < SP:
            pltpu.sync_copy(zslice, src_sh.at[pl.ds(nbase, SLICE)])
        plsc.subcore_barrier()

    out_sh = cb_sh if (SP - 1) % 2 == 0 else ca_sh
    pltpu.sync_copy(out_sh.at[pl.ds(nbase, SLICE)], c_out.at[pl.ds(nbase, SLICE)])


_prop_step = pl.kernel(
    _prop_body,
    out_type=jax.ShapeDtypeStruct((NPAD,), jnp.float32),
    mesh=_MESH,
    scratch_types=[
        pltpu.VMEM((CHB,), jnp.int32),
        pltpu.VMEM((CHB,), jnp.int32),
        pltpu.VMEM((CHB,), jnp.float32),
        pltpu.VMEM((CHB,), jnp.float32),
        pltpu.VMEM((CHB,), jnp.float32),
        pltpu.VMEM((SLICE,), jnp.float32),
        pltpu.VMEM((16,), jnp.float32),
        pltpu.VMEM((16,), jnp.int32),
        pltpu.VMEM_SHARED((NPAD,), jnp.float32),
        pltpu.VMEM_SHARED((NPAD,), jnp.float32),
    ],
)


def kernel(nodes, edge_index, edge_attr, valid, r, fx, W1, b1, W2, b2, W3, b3):
    N = nodes.shape[0]
    src = edge_index[0].astype(jnp.int32)
    dst = edge_index[1].astype(jnp.int32)

    ea_r = edge_attr.reshape(800, 1000, 16)
    logitsE = _edge_logits(
        ea_r, W1.T, b1.reshape(1, 16), W2.T, b2.reshape(1, 16),
        W3.reshape(1, 16), b3.reshape(1, 1)).reshape(N_EDGES)

    vf = valid[0].astype(jnp.float32)
    validT_pad = jnp.zeros((8, NPAD), jnp.float32).at[:, :N].set(vf.T)

    hop0 = jnp.full((NPAD,), BIGV, jnp.int32).at[0].set(0)

    def _bfs_cond(carry):
        _, _, cont = carry
        return cont > 0

    def _bfs_xla(carry):
        hop, t, _ = carry
        t16 = jnp.full((16,), t, jnp.int32)
        hop_new = _bfs_step(src, dst, hop, t16)
        cur = jnp.max(jnp.where(hop_new[:N_NODES] >= BIGV, -1, hop_new[:N_NODES]))
        cont = (cur == t + SB - 1).astype(jnp.int32)
        return hop_new, t + SB, cont

    hop, _, _ = lax.while_loop(
        _bfs_cond, _bfs_xla, (hop0, jnp.int32(1), jnp.int32(1)))
    max_hop = jnp.max(jnp.where(hop[:N_NODES] >= BIGV, -1, hop[:N_NODES]))

    alpha = _alpha_kernel(src, dst, logitsE, hop, validT_pad)[:N_EDGES]

    c0 = jnp.zeros((NPAD,), jnp.float32).at[0].set(1.0)

    def _prop_xla(_, c):
        return _prop_step(src, dst, alpha, c)

    n_prop = (max_hop + SP - 1) // SP
    c = lax.fori_loop(0, n_prop, _prop_xla, c0)[:N]

    x32 = nodes.reshape(N, 32)
    m32 = jnp.repeat(vf, 4, axis=1)
    return _weighted_sum(c.reshape(N, 1), x32, m32)
